# baseline (speedup 1.0000x reference)
import functools

import jax
import jax.numpy as jnp
from jax import lax
from jax.experimental import pallas as pl
from jax.experimental.pallas import tpu as pltpu
from jax.experimental.pallas import tpu_sc as plsc

_GROUPS = 8
_SEARCH = 9
_LANES = 16


def _make_kernel(N, C, H, W):
    HW = H * W
    PW = W + 2
    PH = H + 2
    Cg = C // _GROUPS
    PB = HW // _LANES
    TB = N * PB
    NW = 32

    mesh = plsc.VectorSubcoreMesh(core_axis_name="c", subcore_axis_name="s")

    @functools.partial(
        pl.kernel,
        out_type=jax.ShapeDtypeStruct((N, _GROUPS * _SEARCH, HW), jnp.float32),
        mesh=mesh,
        compiler_params=pltpu.CompilerParams(
            use_tc_tiling_on_sc=False, needs_layout_passes=False),
        scratch_types=[
            pltpu.VMEM((C, _LANES), jnp.float32),
            pltpu.VMEM((2 * _SEARCH, _LANES), jnp.float32),
            pltpu.VMEM((4, 4 * _LANES), jnp.int32),
            pltpu.VMEM((4, 4 * _LANES, C // 2), jnp.int32),
            pltpu.VMEM((_GROUPS * _SEARCH, _LANES), jnp.float32),
            pltpu.SemaphoreType.DMA,
            pltpu.SemaphoreType.DMA,
            pltpu.SemaphoreType.DMA,
            pltpu.SemaphoreType.DMA,
        ],
    )
    def kern(table, lf, eo, out, lf_v, eo_v, idx_v, rows_v, out_v,
             sem0, sem1, sem2, sem3):
        wid = lax.axis_index("s") * 2 + lax.axis_index("c")
        iota = lax.iota(jnp.int32, _LANES)
        sems = (sem0, sem1, sem2, sem3)

        nblk = (TB - wid + NW - 1) // NW

        def block_body(i, carry):
            del carry
            b = wid + i * NW
            n = b // PB
            p0 = (b % PB) * _LANES

            pltpu.sync_copy(lf.at[n, :, pl.ds(p0, _LANES)], lf_v)
            pltpu.sync_copy(eo.at[n, :, pl.ds(p0, _LANES)], eo_v)

            p = iota + p0
            xi = lax.rem(p, jnp.full((_LANES,), W, jnp.int32))
            yi = lax.div(p, jnp.full((_LANES,), W, jnp.int32))
            xf = xi.astype(jnp.float32)
            yf = yi.astype(jnp.float32)
            nrow = n * (PH * PW)

            def fill_idx(s, buf):
                bx = float(s // 3 - 1)
                by = float(s % 3 - 1)
                x = xf + (bx + eo_v[2 * s, :])
                y = yf + (by + eo_v[2 * s + 1, :])
                xt = x.astype(jnp.int32)
                x0 = jnp.where(xt.astype(jnp.float32) > x, xt - 1, xt)
                yt = y.astype(jnp.int32)
                y0 = jnp.where(yt.astype(jnp.float32) > y, yt - 1, yt)
                fx = x - x0.astype(jnp.float32)
                fy = y - y0.astype(jnp.float32)
                xc0 = jnp.minimum(jnp.maximum(x0 + 1, 0), PW - 1)
                xc1 = jnp.minimum(jnp.maximum(x0 + 2, 0), PW - 1)
                yc0 = jnp.minimum(jnp.maximum(y0 + 1, 0), PH - 1)
                yc1 = jnp.minimum(jnp.maximum(y0 + 2, 0), PH - 1)
                r0 = nrow + yc0 * PW
                r1 = nrow + yc1 * PW
                idx_v[buf, pl.ds(0, _LANES)] = r0 + xc0
                idx_v[buf, pl.ds(_LANES, _LANES)] = r1 + xc0
                idx_v[buf, pl.ds(2 * _LANES, _LANES)] = r0 + xc1
                idx_v[buf, pl.ds(3 * _LANES, _LANES)] = r1 + xc1
                return fx, fy

            rowk = [iota + k * _LANES for k in range(4)]

            NRING = 4
            desc = [None] * NRING
            wlist = [None] * _SEARCH
            for j in range(NRING - 1):
                wlist[j] = fill_idx(j, j)
                desc[j] = pltpu.async_copy(
                    table.at[idx_v.at[j]], rows_v.at[j], sems[j])

            for s in range(_SEARCH):
                buf = s % NRING
                if s + NRING - 1 < _SEARCH:
                    sn = s + NRING - 1
                    bn = sn % NRING
                    wlist[sn] = fill_idx(sn, bn)
                    desc[bn] = pltpu.async_copy(
                        table.at[idx_v.at[bn]], rows_v.at[bn], sems[bn])
                fx, fy = wlist[s]
                gx = 1.0 - fx
                gy = 1.0 - fy
                wa = plsc.pack(gx * gy, gx * gy,
                               format=plsc.PackFormat.INTERLEAVED,
                               preferred_element_type=jnp.bfloat16)
                wb = plsc.pack(gx * fy, gx * fy,
                               format=plsc.PackFormat.INTERLEAVED,
                               preferred_element_type=jnp.bfloat16)
                wc = plsc.pack(fx * gy, fx * gy,
                               format=plsc.PackFormat.INTERLEAVED,
                               preferred_element_type=jnp.bfloat16)
                wd = plsc.pack(fx * fy, fx * fy,
                               format=plsc.PackFormat.INTERLEAVED,
                               preferred_element_type=jnp.bfloat16)
                desc[buf].wait()
                rbuf = rows_v.at[buf]

                for g in range(_GROUPS):
                    def ch_body(cq, acc, g=g, rbuf=rbuf, wa=wa, wb=wb, wc=wc,
                                wd=wd):
                        for u in range(2):
                            w0 = cq * 2 + u
                            col = ((iota + w0) & (Cg // 2 - 1)) + g * (Cg // 2)
                            va = plsc.bitcast(
                                plsc.load_gather(rbuf, [rowk[0], col]),
                                jnp.bfloat16)
                            vb = plsc.bitcast(
                                plsc.load_gather(rbuf, [rowk[1], col]),
                                jnp.bfloat16)
                            vc = plsc.bitcast(
                                plsc.load_gather(rbuf, [rowk[2], col]),
                                jnp.bfloat16)
                            vd = plsc.bitcast(
                                plsc.load_gather(rbuf, [rowk[3], col]),
                                jnp.bfloat16)
                            ch0 = col * 2
                            lf0 = plsc.load_gather(lf_v, [ch0, iota])
                            lf1 = plsc.load_gather(lf_v, [ch0 + 1, iota])
                            sv = va * wa + vb * wb + vc * wc + vd * wd
                            sv0, sv1 = plsc.unpack(
                                sv, format=plsc.PackFormat.INTERLEAVED,
                                preferred_element_type=jnp.float32)
                            acc = acc + sv0 * lf0
                            acc = acc + sv1 * lf1
                        return acc

                    acc = plsc.parallel_loop(
                        0, Cg // 4, unroll=2,
                        carry=jnp.zeros((_LANES,), jnp.float32))(ch_body)
                    out_v[g * _SEARCH + s, :] = acc * (1.0 / Cg)

            pltpu.sync_copy(out_v, out.at[n, :, pl.ds(p0, _LANES)])
            return 0

        lax.fori_loop(0, nblk, block_body, 0)

    return kern


def kernel(left_feature, right_feature, extra_offset):
    N, C, H, W = left_feature.shape
    table = jax.lax.bitcast_convert_type(
        jnp.pad(
            right_feature.transpose(0, 2, 3, 1),
            ((0, 0), (1, 1), (1, 1), (0, 0)),
        ).astype(jnp.bfloat16).reshape(N * (H + 2) * (W + 2), C // 2, 2),
        jnp.int32)
    lf = left_feature.reshape(N, C, H * W)
    eo = extra_offset.reshape(N, 2 * _SEARCH, H * W)
    out = _make_kernel(N, C, H, W)(table, lf, eo)
    return out.reshape(N, _GROUPS * _SEARCH, H, W)


# speedup vs baseline: 1.1442x; 1.1442x over previous
import functools

import jax
import jax.numpy as jnp
from jax import lax
from jax.experimental import pallas as pl
from jax.experimental.pallas import tpu as pltpu
from jax.experimental.pallas import tpu_sc as plsc

_GROUPS = 8
_SEARCH = 9
_LANES = 16


def _make_kernel(N, C, H, W):
    HW = H * W
    PW = W + 2
    PH = H + 2
    Cg = C // _GROUPS
    PB = HW // _LANES
    TB = N * PB
    NW = 32

    mesh = plsc.VectorSubcoreMesh(core_axis_name="c", subcore_axis_name="s")

    @functools.partial(
        pl.kernel,
        out_type=jax.ShapeDtypeStruct((N, _GROUPS * _SEARCH, HW), jnp.float32),
        mesh=mesh,
        compiler_params=pltpu.CompilerParams(
            use_tc_tiling_on_sc=False, needs_layout_passes=False),
        scratch_types=[
            pltpu.VMEM((C, _LANES), jnp.float32),
            pltpu.VMEM((2 * _SEARCH, _LANES), jnp.float32),
            pltpu.VMEM((4, 4 * _LANES), jnp.int32),
            pltpu.VMEM((4, 4 * _LANES, C // 2), jnp.int32),
            pltpu.VMEM((_GROUPS * _SEARCH, _LANES), jnp.float32),
            pltpu.SemaphoreType.DMA,
            pltpu.SemaphoreType.DMA,
            pltpu.SemaphoreType.DMA,
            pltpu.SemaphoreType.DMA,
        ],
    )
    def kern(table, lf, eo, out, lf_v, eo_v, idx_v, rows_v, out_v,
             sem0, sem1, sem2, sem3):
        wid = lax.axis_index("s") * 2 + lax.axis_index("c")
        iota = lax.iota(jnp.int32, _LANES)
        sems = (sem0, sem1, sem2, sem3)

        nblk = (TB - wid + NW - 1) // NW

        def block_body(i, carry):
            del carry
            b = wid + i * NW
            n = b // PB
            p0 = (b % PB) * _LANES

            pltpu.sync_copy(lf.at[n, :, pl.ds(p0, _LANES)], lf_v)
            pltpu.sync_copy(eo.at[n, :, pl.ds(p0, _LANES)], eo_v)

            p = iota + p0
            xi = lax.rem(p, jnp.full((_LANES,), W, jnp.int32))
            yi = lax.div(p, jnp.full((_LANES,), W, jnp.int32))
            xf = xi.astype(jnp.float32)
            yf = yi.astype(jnp.float32)
            nrow = n * (PH * PW)

            def fill_idx(s, buf):
                bx = float(s // 3 - 1)
                by = float(s % 3 - 1)
                x = xf + (bx + eo_v[2 * s, :])
                y = yf + (by + eo_v[2 * s + 1, :])
                xt = x.astype(jnp.int32)
                x0 = jnp.where(xt.astype(jnp.float32) > x, xt - 1, xt)
                yt = y.astype(jnp.int32)
                y0 = jnp.where(yt.astype(jnp.float32) > y, yt - 1, yt)
                fx = x - x0.astype(jnp.float32)
                fy = y - y0.astype(jnp.float32)
                xc0 = jnp.minimum(jnp.maximum(x0 + 1, 0), PW - 1)
                xc1 = jnp.minimum(jnp.maximum(x0 + 2, 0), PW - 1)
                yc0 = jnp.minimum(jnp.maximum(y0 + 1, 0), PH - 1)
                yc1 = jnp.minimum(jnp.maximum(y0 + 2, 0), PH - 1)
                r0 = nrow + yc0 * PW
                r1 = nrow + yc1 * PW
                idx_v[buf, pl.ds(0, _LANES)] = r0 + xc0
                idx_v[buf, pl.ds(_LANES, _LANES)] = r1 + xc0
                idx_v[buf, pl.ds(2 * _LANES, _LANES)] = r0 + xc1
                idx_v[buf, pl.ds(3 * _LANES, _LANES)] = r1 + xc1
                return fx, fy

            rowk = [iota + k * _LANES for k in range(4)]

            NRING = 4
            desc = [None] * NRING
            wlist = [None] * _SEARCH
            for j in range(NRING - 1):
                wlist[j] = fill_idx(j, j)
                desc[j] = pltpu.async_copy(
                    table.at[idx_v.at[j]], rows_v.at[j], sems[j])

            for s in range(_SEARCH):
                buf = s % NRING
                if s + NRING - 1 < _SEARCH:
                    sn = s + NRING - 1
                    bn = sn % NRING
                    wlist[sn] = fill_idx(sn, bn)
                    desc[bn] = pltpu.async_copy(
                        table.at[idx_v.at[bn]], rows_v.at[bn], sems[bn])
                fx, fy = wlist[s]
                gx = 1.0 - fx
                gy = 1.0 - fy
                wa = plsc.pack(gx * gy, gx * gy,
                               format=plsc.PackFormat.INTERLEAVED,
                               preferred_element_type=jnp.bfloat16)
                wb = plsc.pack(gx * fy, gx * fy,
                               format=plsc.PackFormat.INTERLEAVED,
                               preferred_element_type=jnp.bfloat16)
                wc = plsc.pack(fx * gy, fx * gy,
                               format=plsc.PackFormat.INTERLEAVED,
                               preferred_element_type=jnp.bfloat16)
                wd = plsc.pack(fx * fy, fx * fy,
                               format=plsc.PackFormat.INTERLEAVED,
                               preferred_element_type=jnp.bfloat16)
                desc[buf].wait()
                rbuf = rows_v.at[buf]

                for g in range(_GROUPS):
                    def ch_body(cq, acc, g=g, rbuf=rbuf, wa=wa, wb=wb, wc=wc,
                                wd=wd):
                        for u in range(2):
                            w0 = cq * 2 + u
                            col = ((iota + w0) & (Cg // 2 - 1)) + g * (Cg // 2)
                            va = plsc.bitcast(
                                plsc.load_gather(rbuf, [rowk[0], col]),
                                jnp.bfloat16)
                            vb = plsc.bitcast(
                                plsc.load_gather(rbuf, [rowk[1], col]),
                                jnp.bfloat16)
                            vc = plsc.bitcast(
                                plsc.load_gather(rbuf, [rowk[2], col]),
                                jnp.bfloat16)
                            vd = plsc.bitcast(
                                plsc.load_gather(rbuf, [rowk[3], col]),
                                jnp.bfloat16)
                            ch0 = col * 2
                            lf0 = plsc.load_gather(lf_v, [ch0, iota])
                            lf1 = plsc.load_gather(lf_v, [ch0 + 1, iota])
                            sv = va * wa + vb * wb + vc * wc + vd * wd
                            sv0, sv1 = plsc.unpack(
                                sv, format=plsc.PackFormat.INTERLEAVED,
                                preferred_element_type=jnp.float32)
                            acc = acc + sv0 * lf0
                            acc = acc + sv1 * lf1
                        return acc

                    acc = plsc.parallel_loop(
                        0, Cg // 4,
                        carry=jnp.zeros((_LANES,), jnp.float32))(ch_body)
                    out_v[g * _SEARCH + s, :] = acc * (1.0 / Cg)

            pltpu.sync_copy(out_v, out.at[n, :, pl.ds(p0, _LANES)])
            return 0

        lax.fori_loop(0, nblk, block_body, 0)

    return kern


def kernel(left_feature, right_feature, extra_offset):
    N, C, H, W = left_feature.shape
    table = jax.lax.bitcast_convert_type(
        jnp.pad(
            right_feature.transpose(0, 2, 3, 1),
            ((0, 0), (1, 1), (1, 1), (0, 0)),
        ).astype(jnp.bfloat16).reshape(N * (H + 2) * (W + 2), C // 2, 2),
        jnp.int32)
    lf = left_feature.reshape(N, C, H * W)
    eo = extra_offset.reshape(N, 2 * _SEARCH, H * W)
    out = _make_kernel(N, C, H, W)(table, lf, eo)
    return out.reshape(N, _GROUPS * _SEARCH, H, W)
